# Initial kernel scaffold; baseline (speedup 1.0000x reference)
import jax
import jax.numpy as jnp
from jax import lax
from jax.experimental import pallas as pl
from jax.experimental.pallas import tpu as pltpu
from jax.experimental.pallas import tpu_sc as plsc

_B = 4096
_N_SPARSE = 26
_N_DENSE = 13
_VOCAB = 100000
_NC = 2
_NS = 16
_NW = _NC * _NS
_RPW = _B // _NW
_L = 16


def _sc_body(table_hbm, idx_hbm, dense_hbm, wd_hbm, out_hbm,
             idx_v, rows_v, dense_v, wd_v, acc_v, sem):
    wid = lax.axis_index("s") * _NC + lax.axis_index("c")
    base = wid * _RPW

    pltpu.sync_copy(idx_hbm.at[wid], idx_v)
    pltpu.sync_copy(dense_hbm.at[wid], dense_v)
    pltpu.sync_copy(wd_hbm, wd_v)

    for f in range(_N_SPARSE):
        off = f * _VOCAB
        for j in range(_RPW // _L):
            sl = pl.ds(j * _L, _L)
            idx_v[f, sl] = idx_v[f, sl] + off

    chunk = 13
    for c0 in range(0, _N_SPARSE, chunk):
        copies = [
            pltpu.make_async_copy(table_hbm.at[idx_v.at[f]], rows_v.at[f], sem)
            for f in range(c0, c0 + chunk)
        ]
        for cp in copies:
            cp.start()
        for cp in copies:
            cp.wait()

    for j in range(_RPW // _L):
        sl = pl.ds(j * _L, _L)
        acc = rows_v[0, sl]
        for f in range(1, _N_SPARSE):
            acc = acc + rows_v[f, sl]
        for d in range(_N_DENSE):
            acc = acc + dense_v[d, sl] * wd_v[d]
        acc_v[sl] = acc

    pltpu.sync_copy(acc_v, out_hbm.at[pl.ds(base, _RPW)])


@jax.jit
def _run(table, idx_grouped, dense_grouped, wd):
    mesh = plsc.VectorSubcoreMesh(core_axis_name="c", subcore_axis_name="s")
    return pl.kernel(
        _sc_body,
        out_type=jax.ShapeDtypeStruct((_B,), jnp.float32),
        mesh=mesh,
        scratch_types=[
            pltpu.VMEM((_N_SPARSE, _RPW), jnp.int32),
            pltpu.VMEM((_N_SPARSE, _RPW), jnp.float32),
            pltpu.VMEM((_N_DENSE, _RPW), jnp.float32),
            pltpu.VMEM((_L,), jnp.float32),
            pltpu.VMEM((_RPW,), jnp.float32),
            pltpu.SemaphoreType.DMA,
        ],
    )(table, idx_grouped, dense_grouped, wd)


def kernel(X, W_emb, W_dense):
    idx = X[:, :_N_SPARSE].astype(jnp.int32)
    idx_grouped = idx.reshape(_NW, _RPW, _N_SPARSE).transpose(0, 2, 1)
    dense_grouped = (
        X[:, _N_SPARSE:].reshape(_NW, _RPW, _N_DENSE).transpose(0, 2, 1)
    )
    table = W_emb.reshape(-1)
    wd = jnp.pad(W_dense[:, 0], (0, _L - _N_DENSE))
    out = _run(table, idx_grouped, dense_grouped, wd)
    return out.reshape(_B, 1)


# speedup vs baseline: 1.4098x; 1.4098x over previous
import jax
import jax.numpy as jnp
from jax import lax
from jax.experimental import pallas as pl
from jax.experimental.pallas import tpu as pltpu
from jax.experimental.pallas import tpu_sc as plsc

_B = 4096
_N_SPARSE = 26
_N_DENSE = 13
_VOCAB = 100000
_NC = 2
_NS = 16
_NW = _NC * _NS
_RPW = _B // _NW
_L = 16


def _sc_body(table_hbm, idx_hbm, dense_hbm, wd_hbm, out_hbm,
             idx_v, rows_v, dense_v, wd_v, acc_v, sem):
    wid = lax.axis_index("s") * _NC + lax.axis_index("c")
    base = wid * _RPW

    pltpu.sync_copy(idx_hbm.at[wid], idx_v)
    pltpu.sync_copy(dense_hbm.at[wid], dense_v)
    pltpu.sync_copy(wd_hbm, wd_v)

    for f in range(_N_SPARSE):
        off = f * _VOCAB
        for j in range(_RPW // _L):
            sl = pl.ds(j * _L, _L)
            idx_v[f, sl] = idx_v[f, sl] + off

    chunk = 13
    for c0 in range(0, _N_SPARSE, chunk):
        copies = [
            pltpu.make_async_copy(table_hbm.at[idx_v.at[f]], rows_v.at[f], sem)
            for f in range(c0, c0 + chunk)
        ]
        for cp in copies:
            cp.start()
        for cp in copies:
            cp.wait()

    wdv = wd_v[:]
    for j in range(_RPW // _L):
        sl = pl.ds(j * _L, _L)
        acc = rows_v[0, sl]
        for f in range(1, _N_SPARSE):
            acc = acc + rows_v[f, sl]
        for d in range(_N_DENSE):
            acc = acc + dense_v[d, sl] * wdv[d]
        acc_v[sl] = acc

    pltpu.sync_copy(acc_v, out_hbm.at[pl.ds(base, _RPW)])


@jax.jit
def _run(table, idx_grouped, dense_grouped, wd):
    mesh = plsc.VectorSubcoreMesh(core_axis_name="c", subcore_axis_name="s")
    return pl.kernel(
        _sc_body,
        out_type=jax.ShapeDtypeStruct((_B,), jnp.float32),
        mesh=mesh,
        scratch_types=[
            pltpu.VMEM((_N_SPARSE, _RPW), jnp.int32),
            pltpu.VMEM((_N_SPARSE, _RPW), jnp.float32),
            pltpu.VMEM((_N_DENSE, _RPW), jnp.float32),
            pltpu.VMEM((_L,), jnp.float32),
            pltpu.VMEM((_RPW,), jnp.float32),
            pltpu.SemaphoreType.DMA,
        ],
    )(table, idx_grouped, dense_grouped, wd)


def kernel(X, W_emb, W_dense):
    idx = X[:, :_N_SPARSE].astype(jnp.int32)
    idx_grouped = idx.reshape(_NW, _RPW, _N_SPARSE).transpose(0, 2, 1)
    dense_grouped = (
        X[:, _N_SPARSE:].reshape(_NW, _RPW, _N_DENSE).transpose(0, 2, 1)
    )
    table = W_emb.reshape(-1)
    wd = jnp.pad(W_dense[:, 0], (0, _L - _N_DENSE))
    out = _run(table, idx_grouped, dense_grouped, wd)
    return out.reshape(_B, 1)
